# Initial kernel scaffold; baseline (speedup 1.0000x reference)
import jax
import jax.numpy as jnp
from jax.experimental import pallas as pl


def kernel(flat_idx, segment_ids, s_idx, r_idx, ent_embeds, rel_embeds, W, b, v_s):
    raise NotImplementedError("write your pallas kernel here")


# speedup vs baseline: 5.9507x; 5.9507x over previous
import functools

import jax
import jax.numpy as jnp
from jax import lax
from jax.experimental import pallas as pl
from jax.experimental.pallas import tpu as pltpu
from jax.experimental.pallas import tpu_sc as plsc

H = 512
SEQ_LEN = 10
B = 16
N_SEG = B * SEQ_LEN
NSEGP = 256
T = 16384
TBLK = 512
NUM_TILES = T // TBLK

SC_NC = 2
SC_NS = 16
SC_NW = SC_NC * SC_NS
ROWS_PER_W = T // SC_NW
CHUNK = 128
SEG_PER_W = NSEGP // SC_NW


def _sc_gather(flat_idx, s_idx_pad, r_idx_pad, ent_embeds, rel_embeds):
    mesh = plsc.VectorSubcoreMesh(core_axis_name="c", subcore_axis_name="s")

    @functools.partial(
        pl.kernel,
        mesh=mesh,
        out_type=(
            jax.ShapeDtypeStruct((T, H), jnp.float32),
            jax.ShapeDtypeStruct((NSEGP, H), jnp.float32),
            jax.ShapeDtypeStruct((NSEGP, H), jnp.float32),
        ),
        scratch_types=[
            pltpu.VMEM((CHUNK,), jnp.int32),
            pltpu.VMEM((CHUNK, H), jnp.float32),
            pltpu.VMEM((SEG_PER_W,), jnp.int32),
            pltpu.VMEM((SEG_PER_W, H), jnp.float32),
            pltpu.SemaphoreType.DMA,
        ],
    )
    def gather_k(flat_idx_h, sidx_h, ridx_h, ent_h, rel_h,
                 em_o, ss_o, rr_o, idx_v, rows_v, idx_s, rows_s, sem):
        wid = lax.axis_index("s") * SC_NC + lax.axis_index("c")
        for c in range(ROWS_PER_W // CHUNK):
            base = wid * ROWS_PER_W + c * CHUNK
            pltpu.sync_copy(flat_idx_h.at[pl.ds(base, CHUNK)], idx_v)
            pltpu.async_copy(ent_h.at[idx_v], rows_v, sem).wait()
            pltpu.sync_copy(rows_v, em_o.at[pl.ds(base, CHUNK)])
        sb = wid * SEG_PER_W
        pltpu.sync_copy(sidx_h.at[pl.ds(sb, SEG_PER_W)], idx_s)
        pltpu.async_copy(ent_h.at[idx_s], rows_s, sem).wait()
        pltpu.sync_copy(rows_s, ss_o.at[pl.ds(sb, SEG_PER_W)])
        pltpu.sync_copy(ridx_h.at[pl.ds(sb, SEG_PER_W)], idx_s)
        pltpu.async_copy(rel_h.at[idx_s], rows_s, sem).wait()
        pltpu.sync_copy(rows_s, rr_o.at[pl.ds(sb, SEG_PER_W)])

    return gather_k(flat_idx, s_idx_pad, r_idx_pad, ent_embeds, rel_embeds)


def _attn_body(em_ref, seg_ref, ss_ref, rr_ref, w_ref, b_ref, v_ref,
               agg_o, ss_o, rr_o, bias_s, num_s, den_s):
    i = pl.program_id(0)

    @pl.when(i == 0)
    def _init():
        bias_s[...] = (
            jnp.dot(ss_ref[...], w_ref[H:2 * H, :],
                    preferred_element_type=jnp.float32)
            + jnp.dot(rr_ref[...], w_ref[2 * H:3 * H, :],
                      preferred_element_type=jnp.float32)
            + b_ref[...])
        num_s[...] = jnp.zeros_like(num_s)
        den_s[...] = jnp.zeros_like(den_s)

    em = em_ref[...]
    seg = seg_ref[...]
    onehot = (seg == lax.broadcasted_iota(
        jnp.int32, (TBLK, NSEGP), 1)).astype(jnp.float32)
    bias_tok = jnp.dot(onehot, bias_s[...],
                       preferred_element_type=jnp.float32)
    z = jnp.dot(em, w_ref[0:H, :],
                preferred_element_type=jnp.float32) + bias_tok
    za = jnp.tanh(z)
    v = v_ref[...]
    s = jnp.sum(za * v, axis=1, keepdims=True)
    m_bound = jnp.sum(jnp.abs(v))
    e = jnp.exp(s - m_bound)
    num_s[...] += lax.dot_general(onehot, e * em, (((0,), (0,)), ((), ())),
                                  preferred_element_type=jnp.float32)
    den_s[...] += lax.dot_general(onehot, e, (((0,), (0,)), ((), ())),
                                  preferred_element_type=jnp.float32)

    @pl.when(i == pl.num_programs(0) - 1)
    def _fin():
        den = den_s[...]
        mask = (den > 0).astype(jnp.float32)
        agg = num_s[...] / jnp.maximum(den, 1e-37)
        agg_o[...] = agg * mask
        ss_o[...] = ss_ref[...] * mask
        rr_o[...] = rr_ref[...] * mask


def _tc_attn(em, seg_col, ss_pad, rr_pad, W, b2, v2):
    out = pl.pallas_call(
        _attn_body,
        grid=(NUM_TILES,),
        in_specs=[
            pl.BlockSpec((TBLK, H), lambda i: (i, 0)),
            pl.BlockSpec((TBLK, 1), lambda i: (i, 0)),
            pl.BlockSpec((NSEGP, H), lambda i: (0, 0)),
            pl.BlockSpec((NSEGP, H), lambda i: (0, 0)),
            pl.BlockSpec((3 * H, H), lambda i: (0, 0)),
            pl.BlockSpec((1, H), lambda i: (0, 0)),
            pl.BlockSpec((1, H), lambda i: (0, 0)),
        ],
        out_specs=[
            pl.BlockSpec((NSEGP, H), lambda i: (0, 0)),
            pl.BlockSpec((NSEGP, H), lambda i: (0, 0)),
            pl.BlockSpec((NSEGP, H), lambda i: (0, 0)),
        ],
        out_shape=[
            jax.ShapeDtypeStruct((NSEGP, H), jnp.float32),
            jax.ShapeDtypeStruct((NSEGP, H), jnp.float32),
            jax.ShapeDtypeStruct((NSEGP, H), jnp.float32),
        ],
        scratch_shapes=[
            pltpu.VMEM((NSEGP, H), jnp.float32),
            pltpu.VMEM((NSEGP, H), jnp.float32),
            pltpu.VMEM((NSEGP, 1), jnp.float32),
        ],
    )(em, seg_col, ss_pad, rr_pad, W, b2, v2)
    return out


def kernel(flat_idx, segment_ids, s_idx, r_idx, ent_embeds, rel_embeds,
           W, b, v_s):
    flat_idx = flat_idx.astype(jnp.int32)
    s_idx_pad = jnp.pad(s_idx.astype(jnp.int32), (0, NSEGP - N_SEG))
    r_idx_pad = jnp.pad(r_idx.astype(jnp.int32), (0, NSEGP - N_SEG))

    em, ss_pad, rr_pad = _sc_gather(flat_idx, s_idx_pad, r_idx_pad,
                                    ent_embeds, rel_embeds)

    seg_col = segment_ids.astype(jnp.int32).reshape(T, 1)
    b2 = b.reshape(1, H)
    v2 = v_s.reshape(1, H)
    agg, ssm, rrm = _tc_attn(em, seg_col, ss_pad, rr_pad, W, b2, v2)

    row = jnp.concatenate([agg, ssm, rrm], axis=1)[:N_SEG]
    return row.reshape(B, SEQ_LEN, 3 * H)


# speedup vs baseline: 5.9709x; 1.0034x over previous
import functools

import jax
import jax.numpy as jnp
from jax import lax
from jax.experimental import pallas as pl
from jax.experimental.pallas import tpu as pltpu
from jax.experimental.pallas import tpu_sc as plsc

H = 512
SEQ_LEN = 10
B = 16
N_SEG = B * SEQ_LEN
NSEGP = 256
T = 16384
TBLK = 512
NUM_TILES = T // TBLK

SC_NC = 2
SC_NS = 16
SC_NW = SC_NC * SC_NS
ROWS_PER_W = T // SC_NW
CHUNK = 128
SEG_PER_W = NSEGP // SC_NW


def _sc_gather(flat_idx, s_idx_pad, r_idx_pad, ent_embeds, rel_embeds):
    mesh = plsc.VectorSubcoreMesh(core_axis_name="c", subcore_axis_name="s")

    @functools.partial(
        pl.kernel,
        mesh=mesh,
        out_type=(
            jax.ShapeDtypeStruct((T, H), jnp.float32),
            jax.ShapeDtypeStruct((NSEGP, H), jnp.float32),
            jax.ShapeDtypeStruct((NSEGP, H), jnp.float32),
        ),
        scratch_types=[
            pltpu.VMEM((CHUNK,), jnp.int32),
            pltpu.VMEM((CHUNK, H), jnp.float32),
            pltpu.VMEM((SEG_PER_W,), jnp.int32),
            pltpu.VMEM((SEG_PER_W, H), jnp.float32),
            pltpu.SemaphoreType.DMA,
        ],
    )
    def gather_k(flat_idx_h, sidx_h, ridx_h, ent_h, rel_h,
                 em_o, ss_o, rr_o, idx_v, rows_v, idx_s, rows_s, sem):
        wid = lax.axis_index("s") * SC_NC + lax.axis_index("c")
        for c in range(ROWS_PER_W // CHUNK):
            base = wid * ROWS_PER_W + c * CHUNK
            pltpu.sync_copy(flat_idx_h.at[pl.ds(base, CHUNK)], idx_v)
            pltpu.async_copy(ent_h.at[idx_v], rows_v, sem).wait()
            pltpu.sync_copy(rows_v, em_o.at[pl.ds(base, CHUNK)])
        sb = wid * SEG_PER_W
        pltpu.sync_copy(sidx_h.at[pl.ds(sb, SEG_PER_W)], idx_s)
        pltpu.async_copy(ent_h.at[idx_s], rows_s, sem).wait()
        pltpu.sync_copy(rows_s, ss_o.at[pl.ds(sb, SEG_PER_W)])
        pltpu.sync_copy(ridx_h.at[pl.ds(sb, SEG_PER_W)], idx_s)
        pltpu.async_copy(rel_h.at[idx_s], rows_s, sem).wait()
        pltpu.sync_copy(rows_s, rr_o.at[pl.ds(sb, SEG_PER_W)])

    return gather_k(flat_idx, s_idx_pad, r_idx_pad, ent_embeds, rel_embeds)


def _attn_body(em_ref, seg_ref, ss_ref, rr_ref, w_ref, b_ref, v_ref,
               agg_o, ss_o, rr_o, bias_s, num_s, den_s):
    i = pl.program_id(0)

    @pl.when(i == 0)
    def _init():
        bias_s[...] = (
            jnp.dot(ss_ref[...], w_ref[H:2 * H, :],
                    preferred_element_type=jnp.float32)
            + jnp.dot(rr_ref[...], w_ref[2 * H:3 * H, :],
                      preferred_element_type=jnp.float32)
            + b_ref[...])
        num_s[...] = jnp.zeros_like(num_s)
        den_s[...] = jnp.zeros_like(den_s)

    em = em_ref[...]
    seg = seg_ref[...]
    onehot = (seg == lax.broadcasted_iota(
        jnp.int32, (TBLK, NSEGP), 1)).astype(jnp.float32)
    ohb = onehot.astype(jnp.bfloat16)
    bias_tok = jnp.dot(ohb, bias_s[...].astype(jnp.bfloat16),
                       preferred_element_type=jnp.float32)
    z = jnp.dot(em.astype(jnp.bfloat16),
                w_ref[0:H, :].astype(jnp.bfloat16),
                preferred_element_type=jnp.float32) + bias_tok
    za = jnp.tanh(z)
    v = v_ref[...]
    s = jnp.sum(za * v, axis=1, keepdims=True)
    m_bound = jnp.sum(jnp.abs(v))
    e = jnp.exp(s - m_bound)
    num_s[...] += lax.dot_general(onehot, e * em, (((0,), (0,)), ((), ())),
                                  preferred_element_type=jnp.float32)
    den_s[...] += lax.dot_general(onehot, e, (((0,), (0,)), ((), ())),
                                  preferred_element_type=jnp.float32)

    @pl.when(i == pl.num_programs(0) - 1)
    def _fin():
        den = den_s[...]
        mask = (den > 0).astype(jnp.float32)
        agg = num_s[...] / jnp.maximum(den, 1e-37)
        agg_o[...] = agg * mask
        ss_o[...] = ss_ref[...] * mask
        rr_o[...] = rr_ref[...] * mask


def _tc_attn(em, seg_col, ss_pad, rr_pad, W, b2, v2):
    out = pl.pallas_call(
        _attn_body,
        grid=(NUM_TILES,),
        in_specs=[
            pl.BlockSpec((TBLK, H), lambda i: (i, 0)),
            pl.BlockSpec((TBLK, 1), lambda i: (i, 0)),
            pl.BlockSpec((NSEGP, H), lambda i: (0, 0)),
            pl.BlockSpec((NSEGP, H), lambda i: (0, 0)),
            pl.BlockSpec((3 * H, H), lambda i: (0, 0)),
            pl.BlockSpec((1, H), lambda i: (0, 0)),
            pl.BlockSpec((1, H), lambda i: (0, 0)),
        ],
        out_specs=[
            pl.BlockSpec((NSEGP, H), lambda i: (0, 0)),
            pl.BlockSpec((NSEGP, H), lambda i: (0, 0)),
            pl.BlockSpec((NSEGP, H), lambda i: (0, 0)),
        ],
        out_shape=[
            jax.ShapeDtypeStruct((NSEGP, H), jnp.float32),
            jax.ShapeDtypeStruct((NSEGP, H), jnp.float32),
            jax.ShapeDtypeStruct((NSEGP, H), jnp.float32),
        ],
        scratch_shapes=[
            pltpu.VMEM((NSEGP, H), jnp.float32),
            pltpu.VMEM((NSEGP, H), jnp.float32),
            pltpu.VMEM((NSEGP, 1), jnp.float32),
        ],
    )(em, seg_col, ss_pad, rr_pad, W, b2, v2)
    return out


def kernel(flat_idx, segment_ids, s_idx, r_idx, ent_embeds, rel_embeds,
           W, b, v_s):
    flat_idx = flat_idx.astype(jnp.int32)
    s_idx_pad = jnp.pad(s_idx.astype(jnp.int32), (0, NSEGP - N_SEG))
    r_idx_pad = jnp.pad(r_idx.astype(jnp.int32), (0, NSEGP - N_SEG))

    em, ss_pad, rr_pad = _sc_gather(flat_idx, s_idx_pad, r_idx_pad,
                                    ent_embeds, rel_embeds)

    seg_col = segment_ids.astype(jnp.int32).reshape(T, 1)
    b2 = b.reshape(1, H)
    v2 = v_s.reshape(1, H)
    agg, ssm, rrm = _tc_attn(em, seg_col, ss_pad, rr_pad, W, b2, v2)

    row = jnp.concatenate([agg, ssm, rrm], axis=1)[:N_SEG]
    return row.reshape(B, SEQ_LEN, 3 * H)


# speedup vs baseline: 6.1383x; 1.0280x over previous
import functools

import jax
import jax.numpy as jnp
from jax import lax
from jax.experimental import pallas as pl
from jax.experimental.pallas import tpu as pltpu
from jax.experimental.pallas import tpu_sc as plsc

H = 512
SEQ_LEN = 10
B = 16
N_SEG = B * SEQ_LEN
NSEGP = 256
T = 16384
TBLK = 512
NUM_TILES = T // TBLK

SC_NC = 2
SC_NS = 16
SC_NW = SC_NC * SC_NS
ROWS_PER_W = T // SC_NW
CHUNK = 64
NCH = ROWS_PER_W // CHUNK
SEG_PER_W = NSEGP // SC_NW


def _sc_gather(flat_idx, s_idx_pad, r_idx_pad, ent_embeds, rel_embeds):
    mesh = plsc.VectorSubcoreMesh(core_axis_name="c", subcore_axis_name="s")

    @functools.partial(
        pl.kernel,
        mesh=mesh,
        out_type=(
            jax.ShapeDtypeStruct((T, H), jnp.float32),
            jax.ShapeDtypeStruct((NSEGP, H), jnp.float32),
            jax.ShapeDtypeStruct((NSEGP, H), jnp.float32),
        ),
        scratch_types=[
            pltpu.VMEM((ROWS_PER_W,), jnp.int32),
            pltpu.VMEM((CHUNK, H), jnp.float32),
            pltpu.VMEM((CHUNK, H), jnp.float32),
            pltpu.VMEM((SEG_PER_W,), jnp.int32),
            pltpu.VMEM((SEG_PER_W, H), jnp.float32),
            pltpu.SemaphoreType.DMA,
            pltpu.SemaphoreType.DMA,
            pltpu.SemaphoreType.DMA,
            pltpu.SemaphoreType.DMA,
        ],
    )
    def gather_k(flat_idx_h, sidx_h, ridx_h, ent_h, rel_h,
                 em_o, ss_o, rr_o, idx_v, rows0_v, rows1_v, idx_s, rows_s,
                 gsem0, gsem1, ssem0, ssem1, ):
        wid = lax.axis_index("s") * SC_NC + lax.axis_index("c")
        base_w = wid * ROWS_PER_W
        pltpu.sync_copy(flat_idx_h.at[pl.ds(base_w, ROWS_PER_W)], idx_v)
        bufs = (rows0_v, rows1_v)
        gsems = (gsem0, gsem1)
        ssems = (ssem0, ssem1)

        def start_gather(c):
            b = c % 2
            return pltpu.async_copy(
                ent_h.at[idx_v.at[pl.ds(c * CHUNK, CHUNK)]], bufs[b], gsems[b])

        def start_store(c):
            b = c % 2
            return pltpu.async_copy(
                bufs[b], em_o.at[pl.ds(base_w + c * CHUNK, CHUNK)], ssems[b])

        hg = [None] * NCH
        hs = [None] * NCH
        hg[0] = start_gather(0)
        for c in range(NCH):
            if c + 1 < NCH:
                if c >= 1:
                    hs[c - 1].wait()
                hg[c + 1] = start_gather(c + 1)
            hg[c].wait()
            hs[c] = start_store(c)
        hs[NCH - 2].wait()
        hs[NCH - 1].wait()

        sb = wid * SEG_PER_W
        pltpu.sync_copy(sidx_h.at[pl.ds(sb, SEG_PER_W)], idx_s)
        pltpu.async_copy(ent_h.at[idx_s], rows_s, gsem0).wait()
        pltpu.sync_copy(rows_s, ss_o.at[pl.ds(sb, SEG_PER_W)])
        pltpu.sync_copy(ridx_h.at[pl.ds(sb, SEG_PER_W)], idx_s)
        pltpu.async_copy(rel_h.at[idx_s], rows_s, gsem0).wait()
        pltpu.sync_copy(rows_s, rr_o.at[pl.ds(sb, SEG_PER_W)])

    return gather_k(flat_idx, s_idx_pad, r_idx_pad, ent_embeds, rel_embeds)


def _attn_body(em_ref, seg_ref, ss_ref, rr_ref, w_ref, b_ref, v_ref,
               agg_o, ss_o, rr_o, bias_s, num_s, den_s):
    i = pl.program_id(0)

    @pl.when(i == 0)
    def _init():
        bias_s[...] = (
            jnp.dot(ss_ref[...], w_ref[H:2 * H, :],
                    preferred_element_type=jnp.float32)
            + jnp.dot(rr_ref[...], w_ref[2 * H:3 * H, :],
                      preferred_element_type=jnp.float32)
            + b_ref[...])
        num_s[...] = jnp.zeros_like(num_s)
        den_s[...] = jnp.zeros_like(den_s)

    em = em_ref[...]
    seg = seg_ref[...]
    onehot = (seg == lax.broadcasted_iota(
        jnp.int32, (TBLK, NSEGP), 1)).astype(jnp.float32)
    ohb = onehot.astype(jnp.bfloat16)
    bias_tok = jnp.dot(ohb, bias_s[...].astype(jnp.bfloat16),
                       preferred_element_type=jnp.float32)
    z = jnp.dot(em.astype(jnp.bfloat16),
                w_ref[0:H, :].astype(jnp.bfloat16),
                preferred_element_type=jnp.float32) + bias_tok
    za = jnp.tanh(z)
    v = v_ref[...]
    s = jnp.sum(za * v, axis=1, keepdims=True)
    m_bound = jnp.sum(jnp.abs(v))
    e = jnp.exp(s - m_bound)
    num_s[...] += lax.dot_general(onehot, e * em, (((0,), (0,)), ((), ())),
                                  preferred_element_type=jnp.float32)
    den_s[...] += lax.dot_general(onehot, e, (((0,), (0,)), ((), ())),
                                  preferred_element_type=jnp.float32)

    @pl.when(i == pl.num_programs(0) - 1)
    def _fin():
        den = den_s[...]
        mask = (den > 0).astype(jnp.float32)
        agg = num_s[...] / jnp.maximum(den, 1e-37)
        agg_o[...] = agg * mask
        ss_o[...] = ss_ref[...] * mask
        rr_o[...] = rr_ref[...] * mask


def _tc_attn(em, seg_col, ss_pad, rr_pad, W, b2, v2):
    out = pl.pallas_call(
        _attn_body,
        grid=(NUM_TILES,),
        in_specs=[
            pl.BlockSpec((TBLK, H), lambda i: (i, 0)),
            pl.BlockSpec((TBLK, 1), lambda i: (i, 0)),
            pl.BlockSpec((NSEGP, H), lambda i: (0, 0)),
            pl.BlockSpec((NSEGP, H), lambda i: (0, 0)),
            pl.BlockSpec((3 * H, H), lambda i: (0, 0)),
            pl.BlockSpec((1, H), lambda i: (0, 0)),
            pl.BlockSpec((1, H), lambda i: (0, 0)),
        ],
        out_specs=[
            pl.BlockSpec((NSEGP, H), lambda i: (0, 0)),
            pl.BlockSpec((NSEGP, H), lambda i: (0, 0)),
            pl.BlockSpec((NSEGP, H), lambda i: (0, 0)),
        ],
        out_shape=[
            jax.ShapeDtypeStruct((NSEGP, H), jnp.float32),
            jax.ShapeDtypeStruct((NSEGP, H), jnp.float32),
            jax.ShapeDtypeStruct((NSEGP, H), jnp.float32),
        ],
        scratch_shapes=[
            pltpu.VMEM((NSEGP, H), jnp.float32),
            pltpu.VMEM((NSEGP, H), jnp.float32),
            pltpu.VMEM((NSEGP, 1), jnp.float32),
        ],
    )(em, seg_col, ss_pad, rr_pad, W, b2, v2)
    return out


def kernel(flat_idx, segment_ids, s_idx, r_idx, ent_embeds, rel_embeds,
           W, b, v_s):
    flat_idx = flat_idx.astype(jnp.int32)
    s_idx_pad = jnp.pad(s_idx.astype(jnp.int32), (0, NSEGP - N_SEG))
    r_idx_pad = jnp.pad(r_idx.astype(jnp.int32), (0, NSEGP - N_SEG))

    em, ss_pad, rr_pad = _sc_gather(flat_idx, s_idx_pad, r_idx_pad,
                                    ent_embeds, rel_embeds)

    seg_col = segment_ids.astype(jnp.int32).reshape(T, 1)
    b2 = b.reshape(1, H)
    v2 = v_s.reshape(1, H)
    agg, ssm, rrm = _tc_attn(em, seg_col, ss_pad, rr_pad, W, b2, v2)

    row = jnp.concatenate([agg, ssm, rrm], axis=1)[:N_SEG]
    return row.reshape(B, SEQ_LEN, 3 * H)
